# Initial kernel scaffold; baseline (speedup 1.0000x reference)
import functools

import jax
import jax.numpy as jnp
from jax import lax
from jax.experimental import pallas as pl
from jax.experimental.pallas import tpu as pltpu
from jax.experimental.pallas import tpu_sc as plsc

N_NODES = 10000
N_HE = 10000
N_EDGES = 320000
DIM = 128

NC = 2
NS = 16
NW = NC * NS
E_PER_TILE = N_EDGES // NW
CHUNK = 80
N_CHUNKS = E_PER_TILE // CHUNK
ROWS_PER_TILE = N_NODES // NS
ZROWS = 125
DEGW = 16

_MESH = dict(core_axis_name="c", subcore_axis_name="s")


def _seg_sum_partials(values, gather_idx, scatter_idx, n_out):
    mesh = plsc.VectorSubcoreMesh(**_MESH)

    @functools.partial(
        pl.kernel,
        mesh=mesh,
        out_type=jax.ShapeDtypeStruct((NC, n_out, DIM), jnp.float32),
        scratch_types=[
            pltpu.VMEM((CHUNK,), jnp.int32),
            pltpu.VMEM((CHUNK,), jnp.int32),
            pltpu.VMEM((CHUNK, DIM), jnp.float32),
            pltpu.VMEM((ZROWS, DIM), jnp.float32),
            pltpu.VMEM_SHARED((n_out, DIM), jnp.float32),
            pltpu.SemaphoreType.DMA,
        ],
    )
    def k(vals_hbm, gidx_hbm, sidx_hbm, out_hbm, gbuf, sbuf, rows, zbuf, acc, sem):
        cid = lax.axis_index("c")
        sid = lax.axis_index("s")
        wid = sid * NC + cid

        def zrow(r, _):
            for c in range(DIM // 16):
                zbuf[r, pl.ds(c * 16, 16)] = jnp.zeros((16,), jnp.float32)
            return 0

        lax.fori_loop(0, ZROWS, zrow, 0)
        row0 = sid * ROWS_PER_TILE

        def zcp(j, _):
            pltpu.sync_copy(zbuf, acc.at[pl.ds(row0 + j * ZROWS, ZROWS)])
            return 0

        lax.fori_loop(0, ROWS_PER_TILE // ZROWS, zcp, 0)
        plsc.subcore_barrier()

        base = wid * E_PER_TILE

        def step(i, _):
            off = base + i * CHUNK
            pltpu.sync_copy(gidx_hbm.at[pl.ds(off, CHUNK)], gbuf)
            pltpu.sync_copy(sidx_hbm.at[pl.ds(off, CHUNK)], sbuf)
            pltpu.async_copy(vals_hbm.at[gbuf], rows, sem).wait()
            pltpu.sync_copy(rows, acc.at[sbuf], add=True)
            return 0

        lax.fori_loop(0, N_CHUNKS, step, 0)
        plsc.subcore_barrier()

        def wb(j, _):
            r0 = row0 + j * ZROWS
            pltpu.sync_copy(acc.at[pl.ds(r0, ZROWS)],
                            out_hbm.at[cid, pl.ds(r0, ZROWS)])
            return 0

        lax.fori_loop(0, ROWS_PER_TILE // ZROWS, wb, 0)

    return k(values, gather_idx, scatter_idx)


def _degree_partials(src, he):
    mesh = plsc.VectorSubcoreMesh(**_MESH)

    @functools.partial(
        pl.kernel,
        mesh=mesh,
        out_type=[
            jax.ShapeDtypeStruct((NC, N_NODES, DEGW), jnp.float32),
            jax.ShapeDtypeStruct((NC, N_HE, DEGW), jnp.float32),
        ],
        scratch_types=[
            pltpu.VMEM((CHUNK,), jnp.int32),
            pltpu.VMEM((CHUNK,), jnp.int32),
            pltpu.VMEM((CHUNK, DEGW), jnp.float32),
            pltpu.VMEM((ROWS_PER_TILE, DEGW), jnp.float32),
            pltpu.VMEM_SHARED((N_NODES, DEGW), jnp.float32),
            pltpu.VMEM_SHARED((N_HE, DEGW), jnp.float32),
        ],
    )
    def k(src_hbm, he_hbm, outD, outB, sbuf, hbuf, ones, zbuf, accD, accB):
        cid = lax.axis_index("c")
        sid = lax.axis_index("s")
        wid = sid * NC + cid

        def fill_ones(r, _):
            ones[r] = jnp.ones((DEGW,), jnp.float32)
            return 0

        lax.fori_loop(0, CHUNK, fill_ones, 0)

        def fill_zero(r, _):
            zbuf[r] = jnp.zeros((DEGW,), jnp.float32)
            return 0

        lax.fori_loop(0, ROWS_PER_TILE, fill_zero, 0)

        row0 = sid * ROWS_PER_TILE
        pltpu.sync_copy(zbuf, accD.at[pl.ds(row0, ROWS_PER_TILE)])
        pltpu.sync_copy(zbuf, accB.at[pl.ds(row0, ROWS_PER_TILE)])
        plsc.subcore_barrier()

        base = wid * E_PER_TILE

        def step(i, _):
            off = base + i * CHUNK
            pltpu.sync_copy(src_hbm.at[pl.ds(off, CHUNK)], sbuf)
            pltpu.sync_copy(he_hbm.at[pl.ds(off, CHUNK)], hbuf)
            pltpu.sync_copy(ones, accD.at[sbuf], add=True)
            pltpu.sync_copy(ones, accB.at[hbuf], add=True)
            return 0

        lax.fori_loop(0, N_CHUNKS, step, 0)
        plsc.subcore_barrier()

        pltpu.sync_copy(accD.at[pl.ds(row0, ROWS_PER_TILE)],
                        outD.at[cid, pl.ds(row0, ROWS_PER_TILE)])
        pltpu.sync_copy(accB.at[pl.ds(row0, ROWS_PER_TILE)],
                        outB.at[cid, pl.ds(row0, ROWS_PER_TILE)])

    return k(src, he)


BLK = 500


def _matmul(h, W):
    def body(h_ref, w_ref, o_ref):
        o_ref[...] = jnp.dot(h_ref[...], w_ref[...],
                             preferred_element_type=jnp.float32)

    return pl.pallas_call(
        body,
        grid=(N_NODES // BLK,),
        in_specs=[
            pl.BlockSpec((BLK, DIM), lambda i: (i, 0)),
            pl.BlockSpec((DIM, DIM), lambda i: (0, 0)),
        ],
        out_specs=pl.BlockSpec((BLK, DIM), lambda i: (i, 0)),
        out_shape=jax.ShapeDtypeStruct((N_NODES, DIM), jnp.float32),
    )(h, W)


def _inv_deg(d_ref):
    d = d_ref[0, :, 0] + d_ref[1, :, 0]
    return jnp.where(d > 0.0, 1.0 / d, 0.0)


def _combine_scale(parts, deg):
    def body(p_ref, d_ref, o_ref):
        inv = _inv_deg(d_ref)
        o_ref[...] = inv[:, None] * (p_ref[0] + p_ref[1])

    return pl.pallas_call(
        body,
        grid=(N_NODES // BLK,),
        in_specs=[
            pl.BlockSpec((NC, BLK, DIM), lambda i: (0, i, 0)),
            pl.BlockSpec((NC, BLK, DEGW), lambda i: (0, i, 0)),
        ],
        out_specs=pl.BlockSpec((BLK, DIM), lambda i: (i, 0)),
        out_shape=jax.ShapeDtypeStruct((N_NODES, DIM), jnp.float32),
    )(parts, deg)


def _combine_mm(parts, deg, b, W):
    def body(p_ref, d_ref, b_ref, w_ref, o_ref):
        inv = _inv_deg(d_ref)
        t = inv[:, None] * (p_ref[0] + p_ref[1]) + b_ref[...][None, :]
        t = jnp.where(t > 0.0, t, jnp.expm1(t))
        o_ref[...] = jnp.dot(t, w_ref[...], preferred_element_type=jnp.float32)

    return pl.pallas_call(
        body,
        grid=(N_NODES // BLK,),
        in_specs=[
            pl.BlockSpec((NC, BLK, DIM), lambda i: (0, i, 0)),
            pl.BlockSpec((NC, BLK, DEGW), lambda i: (0, i, 0)),
            pl.BlockSpec((DIM,), lambda i: (0,)),
            pl.BlockSpec((DIM, DIM), lambda i: (0, 0)),
        ],
        out_specs=pl.BlockSpec((BLK, DIM), lambda i: (i, 0)),
        out_shape=jax.ShapeDtypeStruct((N_NODES, DIM), jnp.float32),
    )(parts, deg, b, W)


def _combine_final(parts, deg, b):
    def body(p_ref, d_ref, b_ref, o_ref):
        inv = _inv_deg(d_ref)
        o_ref[...] = inv[:, None] * (p_ref[0] + p_ref[1]) + b_ref[...][None, :]

    return pl.pallas_call(
        body,
        grid=(N_NODES // BLK,),
        in_specs=[
            pl.BlockSpec((NC, BLK, DIM), lambda i: (0, i, 0)),
            pl.BlockSpec((NC, BLK, DEGW), lambda i: (0, i, 0)),
            pl.BlockSpec((DIM,), lambda i: (0,)),
        ],
        out_specs=pl.BlockSpec((BLK, DIM), lambda i: (i, 0)),
        out_shape=jax.ShapeDtypeStruct((N_NODES, DIM), jnp.float32),
    )(parts, deg, b)


def kernel(x, edges, edge_weight, W1, b1, W2, b2):
    del edge_weight
    src = edges[0]
    he = edges[1]

    degD, degB = _degree_partials(src, he)

    h = _matmul(x, W1)
    pa = _seg_sum_partials(h, src, he, N_HE)
    xe = _combine_scale(pa, degB)
    pb = _seg_sum_partials(xe, he, src, N_NODES)
    h2 = _combine_mm(pb, degD, b1, W2)

    pa2 = _seg_sum_partials(h2, src, he, N_HE)
    xe2 = _combine_scale(pa2, degB)
    pb2 = _seg_sum_partials(xe2, he, src, N_NODES)
    return _combine_final(pb2, degD, b2)


# speedup vs baseline: 7.0673x; 7.0673x over previous
import functools

import jax
import jax.numpy as jnp
from jax import lax
from jax.experimental import pallas as pl
from jax.experimental.pallas import tpu as pltpu
from jax.experimental.pallas import tpu_sc as plsc

N_NODES = 10000
N_HE = 10000
N_EDGES = 320000
DIM = 128

NC = 2
NS = 16
NW = NC * NS
E_PER_TILE = N_EDGES // NW
CHUNK = 80
N_CHUNKS = E_PER_TILE // CHUNK
N_PAD = 10240
ROWS_PER_TILE = N_PAD // NS
ZROWS = 128
DEGW = 16

_MESH = dict(core_axis_name="c", subcore_axis_name="s")


def _seg_sum_partials(values, gather_idx, scatter_idx):
    mesh = plsc.VectorSubcoreMesh(**_MESH)

    @functools.partial(
        pl.kernel,
        mesh=mesh,
        out_type=jax.ShapeDtypeStruct((NC, N_PAD, DIM), jnp.float32),
        scratch_types=[
            pltpu.VMEM((CHUNK,), jnp.int32),
            pltpu.VMEM((CHUNK,), jnp.int32),
            pltpu.VMEM((CHUNK, DIM), jnp.float32),
            pltpu.VMEM((ZROWS, DIM), jnp.float32),
            pltpu.VMEM_SHARED((N_PAD, DIM), jnp.float32),
            pltpu.SemaphoreType.DMA,
        ],
    )
    def k(vals_hbm, gidx_hbm, sidx_hbm, out_hbm, gbuf, sbuf, rows, zbuf, acc, sem):
        cid = lax.axis_index("c")
        sid = lax.axis_index("s")
        wid = sid * NC + cid

        def zrow(r, _):
            for c in range(DIM // 16):
                zbuf[r, pl.ds(c * 16, 16)] = jnp.zeros((16,), jnp.float32)
            return 0

        lax.fori_loop(0, ZROWS, zrow, 0)
        row0 = sid * ROWS_PER_TILE

        def zcp(j, _):
            pltpu.sync_copy(zbuf, acc.at[pl.ds(row0 + j * ZROWS, ZROWS)])
            return 0

        lax.fori_loop(0, ROWS_PER_TILE // ZROWS, zcp, 0)
        plsc.subcore_barrier()

        base = wid * E_PER_TILE

        def step(i, _):
            off = base + i * CHUNK
            pltpu.sync_copy(gidx_hbm.at[pl.ds(off, CHUNK)], gbuf)
            pltpu.sync_copy(sidx_hbm.at[pl.ds(off, CHUNK)], sbuf)
            pltpu.async_copy(vals_hbm.at[gbuf], rows, sem).wait()
            pltpu.sync_copy(rows, acc.at[sbuf], add=True)
            return 0

        lax.fori_loop(0, N_CHUNKS, step, 0)
        plsc.subcore_barrier()

        def wb(j, _):
            r0 = row0 + j * ZROWS
            pltpu.sync_copy(acc.at[pl.ds(r0, ZROWS)], zbuf)
            pltpu.sync_copy(zbuf, out_hbm.at[cid, pl.ds(r0, ZROWS)])
            return 0

        lax.fori_loop(0, ROWS_PER_TILE // ZROWS, wb, 0)

    return k(values, gather_idx, scatter_idx)


def _count_partials(sidx):
    mesh = plsc.VectorSubcoreMesh(**_MESH)

    @functools.partial(
        pl.kernel,
        mesh=mesh,
        out_type=jax.ShapeDtypeStruct((NC, N_PAD, DIM), jnp.float32),
        scratch_types=[
            pltpu.VMEM((CHUNK,), jnp.int32),
            pltpu.VMEM((CHUNK, DIM), jnp.float32),
            pltpu.VMEM((ZROWS, DIM), jnp.float32),
            pltpu.VMEM_SHARED((N_PAD, DIM), jnp.float32),
        ],
    )
    def k(sidx_hbm, out_hbm, sbuf, ones, zbuf, acc):
        cid = lax.axis_index("c")
        sid = lax.axis_index("s")
        wid = sid * NC + cid

        def fill(r, _):
            for c in range(DIM // 16):
                ones[r, pl.ds(c * 16, 16)] = jnp.ones((16,), jnp.float32)
            return 0

        lax.fori_loop(0, CHUNK, fill, 0)

        def zrow(r, _):
            for c in range(DIM // 16):
                zbuf[r, pl.ds(c * 16, 16)] = jnp.zeros((16,), jnp.float32)
            return 0

        lax.fori_loop(0, ZROWS, zrow, 0)
        row0 = sid * ROWS_PER_TILE

        def zcp(j, _):
            pltpu.sync_copy(zbuf, acc.at[pl.ds(row0 + j * ZROWS, ZROWS)])
            return 0

        lax.fori_loop(0, ROWS_PER_TILE // ZROWS, zcp, 0)
        plsc.subcore_barrier()

        base = wid * E_PER_TILE

        def step(i, _):
            off = base + i * CHUNK
            pltpu.sync_copy(sidx_hbm.at[pl.ds(off, CHUNK)], sbuf)
            pltpu.sync_copy(ones, acc.at[sbuf], add=True)
            return 0

        lax.fori_loop(0, N_CHUNKS, step, 0)
        plsc.subcore_barrier()

        def wb(j, _):
            r0 = row0 + j * ZROWS
            pltpu.sync_copy(acc.at[pl.ds(r0, ZROWS)], zbuf)
            pltpu.sync_copy(zbuf, out_hbm.at[cid, pl.ds(r0, ZROWS)])
            return 0

        lax.fori_loop(0, ROWS_PER_TILE // ZROWS, wb, 0)

    return k(sidx)


BLK = 400


def _matmul(h, W):
    def body(h_ref, w_ref, o_ref):
        o_ref[...] = jnp.dot(h_ref[...], w_ref[...],
                             preferred_element_type=jnp.float32)

    return pl.pallas_call(
        body,
        grid=(N_NODES // BLK,),
        in_specs=[
            pl.BlockSpec((BLK, DIM), lambda i: (i, 0)),
            pl.BlockSpec((DIM, DIM), lambda i: (0, 0)),
        ],
        out_specs=pl.BlockSpec((BLK, DIM), lambda i: (i, 0)),
        out_shape=jax.ShapeDtypeStruct((N_NODES, DIM), jnp.float32),
    )(h, W)


def _inv_deg(d_ref):
    d = d_ref[0, :, 0] + d_ref[1, :, 0]
    return jnp.where(d > 0.0, 1.0 / d, 0.0)


def _combine_scale(parts, deg):
    def body(p_ref, d_ref, o_ref):
        inv = _inv_deg(d_ref)
        o_ref[...] = inv[:, None] * (p_ref[0] + p_ref[1])

    return pl.pallas_call(
        body,
        grid=(N_NODES // BLK,),
        in_specs=[
            pl.BlockSpec((NC, BLK, DIM), lambda i: (0, i, 0)),
            pl.BlockSpec((NC, BLK, DIM), lambda i: (0, i, 0)),
        ],
        out_specs=pl.BlockSpec((BLK, DIM), lambda i: (i, 0)),
        out_shape=jax.ShapeDtypeStruct((N_NODES, DIM), jnp.float32),
    )(parts, deg)


def _combine_mm(parts, deg, b, W):
    def body(p_ref, d_ref, b_ref, w_ref, o_ref):
        inv = _inv_deg(d_ref)
        t = inv[:, None] * (p_ref[0] + p_ref[1]) + b_ref[...][None, :]
        t = jnp.where(t > 0.0, t, jnp.exp(jnp.minimum(t, 0.0)) - 1.0)
        o_ref[...] = jnp.dot(t, w_ref[...], preferred_element_type=jnp.float32)

    return pl.pallas_call(
        body,
        grid=(N_NODES // BLK,),
        in_specs=[
            pl.BlockSpec((NC, BLK, DIM), lambda i: (0, i, 0)),
            pl.BlockSpec((NC, BLK, DIM), lambda i: (0, i, 0)),
            pl.BlockSpec((DIM,), lambda i: (0,)),
            pl.BlockSpec((DIM, DIM), lambda i: (0, 0)),
        ],
        out_specs=pl.BlockSpec((BLK, DIM), lambda i: (i, 0)),
        out_shape=jax.ShapeDtypeStruct((N_NODES, DIM), jnp.float32),
    )(parts, deg, b, W)


def _combine_final(parts, deg, b):
    def body(p_ref, d_ref, b_ref, o_ref):
        inv = _inv_deg(d_ref)
        o_ref[...] = inv[:, None] * (p_ref[0] + p_ref[1]) + b_ref[...][None, :]

    return pl.pallas_call(
        body,
        grid=(N_NODES // BLK,),
        in_specs=[
            pl.BlockSpec((NC, BLK, DIM), lambda i: (0, i, 0)),
            pl.BlockSpec((NC, BLK, DIM), lambda i: (0, i, 0)),
            pl.BlockSpec((DIM,), lambda i: (0,)),
        ],
        out_specs=pl.BlockSpec((BLK, DIM), lambda i: (i, 0)),
        out_shape=jax.ShapeDtypeStruct((N_NODES, DIM), jnp.float32),
    )(parts, deg, b)


def kernel(x, edges, edge_weight, W1, b1, W2, b2):
    del edge_weight
    src = edges[0]
    he = edges[1]

    degD = _count_partials(src)
    degB = _count_partials(he)

    h = _matmul(x, W1)
    pa = _seg_sum_partials(h, src, he)
    xe = _combine_scale(pa, degB)
    pb = _seg_sum_partials(xe, he, src)
    h2 = _combine_mm(pb, degD, b1, W2)

    pa2 = _seg_sum_partials(h2, src, he)
    xe2 = _combine_scale(pa2, degB)
    pb2 = _seg_sum_partials(xe2, he, src)
    return _combine_final(pb2, degD, b2)


# speedup vs baseline: 9.1327x; 1.2923x over previous
import functools

import jax
import jax.numpy as jnp
from jax import lax
from jax.experimental import pallas as pl
from jax.experimental.pallas import tpu as pltpu
from jax.experimental.pallas import tpu_sc as plsc

N_NODES = 10000
N_HE = 10000
N_EDGES = 320000
DIM = 128

NC = 2
NS = 16
NW = NC * NS
E_PER_TILE = N_EDGES // NW
CHUNK = 40
N_CHUNKS = E_PER_TILE // CHUNK
K = 2
NB = N_CHUNKS // K
KC = 5
NBC = N_CHUNKS // KC
N_PAD = 10240
ROWS_PER_TILE = N_PAD // NS
ZROWS = 32

_MESH = dict(core_axis_name="c", subcore_axis_name="s")


def _seg_sum_partials(values, idx3):
    mesh = plsc.VectorSubcoreMesh(**_MESH)

    @functools.partial(
        pl.kernel,
        mesh=mesh,
        out_type=jax.ShapeDtypeStruct((NC, N_PAD, DIM), jnp.float32),
        scratch_types=(
            [pltpu.VMEM((2, CHUNK), jnp.int32) for _ in range(2)]
            + [pltpu.VMEM((CHUNK, DIM), jnp.float32) for _ in range(2)]
            + [pltpu.VMEM_SHARED((N_PAD, DIM), jnp.float32)]
            + [pltpu.SemaphoreType.DMA for _ in range(6)]
        ),
    )
    def k(vals_hbm, idx_hbm, out_hbm, *refs):
        idxb = refs[0:2]
        rows = refs[2:4]
        acc = refs[4]
        isem = refs[5:7]
        gsem = refs[7:9]
        ssem = refs[9:11]
        cid = lax.axis_index("c")
        sid = lax.axis_index("s")
        wid = sid * NC + cid

        def zrow(r, _):
            for c in range(DIM // 16):
                rows[0][r, pl.ds(c * 16, 16)] = jnp.zeros((16,), jnp.float32)
            return 0

        lax.fori_loop(0, CHUNK, zrow, 0)
        row0 = sid * ROWS_PER_TILE

        def zcp(j, _):
            pltpu.sync_copy(rows[0], acc.at[pl.ds(row0 + j * CHUNK, CHUNK)])
            return 0

        lax.fori_loop(0, ROWS_PER_TILE // CHUNK, zcp, 0)
        plsc.subcore_barrier()

        def issue_i(p, c):
            pltpu.async_copy(idx_hbm.at[wid, c], idxb[p], isem[p])

        def drain_i(p):
            pltpu.make_async_copy(idx_hbm.at[wid, 0], idxb[p], isem[p]).wait()

        def issue_g(p, _c):
            pltpu.async_copy(vals_hbm.at[idxb[p].at[0]], rows[p], gsem[p])

        def drain_g(p):
            pltpu.make_async_copy(vals_hbm.at[idxb[0].at[0]], rows[p],
                                  gsem[p]).wait()

        def issue_s(p, _c):
            pltpu.async_copy(rows[p], acc.at[idxb[p].at[1]], ssem[p], add=True)

        def drain_s(p):
            pltpu.make_async_copy(rows[p], acc.at[idxb[0].at[1]],
                                  ssem[p]).wait()

        A, B = 0, 1

        issue_i(A, 0)
        drain_i(A)
        issue_g(A, 0)
        issue_i(B, 1)
        drain_i(B)
        issue_g(B, 1)
        drain_g(A)
        issue_s(A, 0)
        drain_s(A)
        issue_i(A, 2)
        drain_g(B)
        issue_s(B, 1)
        drain_i(A)
        issue_g(A, 2)

        def body(j, _):
            cA = 2 * j
            drain_s(B)
            issue_i(B, cA + 1)
            drain_g(A)
            issue_s(A, cA)
            drain_i(B)
            issue_g(B, cA + 1)
            drain_s(A)
            issue_i(A, cA + 2)
            drain_g(B)
            issue_s(B, cA + 1)
            drain_i(A)
            issue_g(A, cA + 2)
            return 0

        lax.fori_loop(1, N_CHUNKS // 2 - 1, body, 0)

        drain_s(B)
        issue_i(B, N_CHUNKS - 1)
        drain_g(A)
        issue_s(A, N_CHUNKS - 2)
        drain_i(B)
        issue_g(B, N_CHUNKS - 1)
        drain_s(A)
        drain_g(B)
        issue_s(B, N_CHUNKS - 1)
        drain_s(B)

        plsc.subcore_barrier()

        def wb(j, _):
            r0 = row0 + j * CHUNK
            pltpu.sync_copy(acc.at[pl.ds(r0, CHUNK)], rows[0])
            pltpu.sync_copy(rows[0], out_hbm.at[cid, pl.ds(r0, CHUNK)])
            return 0

        lax.fori_loop(0, ROWS_PER_TILE // CHUNK, wb, 0)

    return k(values, idx3)


def _count_partials(sidx3):
    mesh = plsc.VectorSubcoreMesh(**_MESH)

    @functools.partial(
        pl.kernel,
        mesh=mesh,
        out_type=jax.ShapeDtypeStruct((NC, N_PAD, DIM), jnp.float32),
        scratch_types=(
            [pltpu.VMEM((N_CHUNKS, CHUNK), jnp.int32),
             pltpu.VMEM((CHUNK, DIM), jnp.float32),
             pltpu.VMEM((ZROWS, DIM), jnp.float32),
             pltpu.VMEM_SHARED((N_PAD, DIM), jnp.float32)]
            + [pltpu.SemaphoreType.DMA for _ in range(KC)]
        ),
    )
    def k(sidx_hbm, out_hbm, sidx2d, ones, zbuf, acc, *sems):
        cid = lax.axis_index("c")
        sid = lax.axis_index("s")
        wid = sid * NC + cid

        def fill(r, _):
            for c in range(DIM // 16):
                ones[r, pl.ds(c * 16, 16)] = jnp.ones((16,), jnp.float32)
            return 0

        lax.fori_loop(0, CHUNK, fill, 0)

        def zrow(r, _):
            for c in range(DIM // 16):
                zbuf[r, pl.ds(c * 16, 16)] = jnp.zeros((16,), jnp.float32)
            return 0

        lax.fori_loop(0, ZROWS, zrow, 0)
        row0 = sid * ROWS_PER_TILE

        def zcp(j, _):
            pltpu.sync_copy(zbuf, acc.at[pl.ds(row0 + j * ZROWS, ZROWS)])
            return 0

        lax.fori_loop(0, ROWS_PER_TILE // ZROWS, zcp, 0)

        pltpu.sync_copy(sidx_hbm.at[wid], sidx2d)
        plsc.subcore_barrier()

        def issue(c, p):
            pltpu.async_copy(ones, acc.at[sidx2d.at[c]], sems[p], add=True)

        def drain(p):
            pltpu.make_async_copy(ones, acc.at[sidx2d.at[0]], sems[p]).wait()

        for p in range(KC):
            issue(p, p)

        def body(j, _):
            for p in range(KC):
                drain(p)
                issue(j * KC + p, p)
            return 0

        lax.fori_loop(1, NBC, body, 0)
        for p in range(KC):
            drain(p)

        plsc.subcore_barrier()

        def wb(j, _):
            r0 = row0 + j * ZROWS
            pltpu.sync_copy(acc.at[pl.ds(r0, ZROWS)], zbuf)
            pltpu.sync_copy(zbuf, out_hbm.at[cid, pl.ds(r0, ZROWS)])
            return 0

        lax.fori_loop(0, ROWS_PER_TILE // ZROWS, wb, 0)

    return k(sidx3)


BLK = 400


def _matmul(h, W):
    def body(h_ref, w_ref, o_ref):
        o_ref[...] = jnp.dot(h_ref[...], w_ref[...],
                             preferred_element_type=jnp.float32)

    return pl.pallas_call(
        body,
        grid=(N_NODES // BLK,),
        in_specs=[
            pl.BlockSpec((BLK, DIM), lambda i: (i, 0)),
            pl.BlockSpec((DIM, DIM), lambda i: (0, 0)),
        ],
        out_specs=pl.BlockSpec((BLK, DIM), lambda i: (i, 0)),
        out_shape=jax.ShapeDtypeStruct((N_NODES, DIM), jnp.float32),
    )(h, W)


def _inv_deg(d_ref):
    d = d_ref[0, :, 0] + d_ref[1, :, 0]
    return jnp.where(d > 0.0, 1.0 / d, 0.0)


def _combine_scale(parts, deg):
    def body(p_ref, d_ref, o_ref):
        inv = _inv_deg(d_ref)
        o_ref[...] = inv[:, None] * (p_ref[0] + p_ref[1])

    return pl.pallas_call(
        body,
        grid=(N_NODES // BLK,),
        in_specs=[
            pl.BlockSpec((NC, BLK, DIM), lambda i: (0, i, 0)),
            pl.BlockSpec((NC, BLK, DIM), lambda i: (0, i, 0)),
        ],
        out_specs=pl.BlockSpec((BLK, DIM), lambda i: (i, 0)),
        out_shape=jax.ShapeDtypeStruct((N_NODES, DIM), jnp.float32),
    )(parts, deg)


def _combine_mm(parts, deg, b, W):
    def body(p_ref, d_ref, b_ref, w_ref, o_ref):
        inv = _inv_deg(d_ref)
        t = inv[:, None] * (p_ref[0] + p_ref[1]) + b_ref[...][None, :]
        t = jnp.where(t > 0.0, t, jnp.exp(jnp.minimum(t, 0.0)) - 1.0)
        o_ref[...] = jnp.dot(t, w_ref[...], preferred_element_type=jnp.float32)

    return pl.pallas_call(
        body,
        grid=(N_NODES // BLK,),
        in_specs=[
            pl.BlockSpec((NC, BLK, DIM), lambda i: (0, i, 0)),
            pl.BlockSpec((NC, BLK, DIM), lambda i: (0, i, 0)),
            pl.BlockSpec((DIM,), lambda i: (0,)),
            pl.BlockSpec((DIM, DIM), lambda i: (0, 0)),
        ],
        out_specs=pl.BlockSpec((BLK, DIM), lambda i: (i, 0)),
        out_shape=jax.ShapeDtypeStruct((N_NODES, DIM), jnp.float32),
    )(parts, deg, b, W)


def _combine_final(parts, deg, b):
    def body(p_ref, d_ref, b_ref, o_ref):
        inv = _inv_deg(d_ref)
        o_ref[...] = inv[:, None] * (p_ref[0] + p_ref[1]) + b_ref[...][None, :]

    return pl.pallas_call(
        body,
        grid=(N_NODES // BLK,),
        in_specs=[
            pl.BlockSpec((NC, BLK, DIM), lambda i: (0, i, 0)),
            pl.BlockSpec((NC, BLK, DIM), lambda i: (0, i, 0)),
            pl.BlockSpec((DIM,), lambda i: (0,)),
        ],
        out_specs=pl.BlockSpec((BLK, DIM), lambda i: (i, 0)),
        out_shape=jax.ShapeDtypeStruct((N_NODES, DIM), jnp.float32),
    )(parts, deg, b)


def kernel(x, edges, edge_weight, W1, b1, W2, b2):
    del edge_weight
    src3 = edges[0].reshape(NW, N_CHUNKS, CHUNK)
    he3 = edges[1].reshape(NW, N_CHUNKS, CHUNK)
    sh = jnp.stack([src3, he3], axis=2)
    hs = jnp.stack([he3, src3], axis=2)

    degD = _count_partials(src3)
    degB = _count_partials(he3)

    h = _matmul(x, W1)
    pa = _seg_sum_partials(h, sh)
    xe = _combine_scale(pa, degB)
    pb = _seg_sum_partials(xe, hs)
    h2 = _combine_mm(pb, degD, b1, W2)

    pa2 = _seg_sum_partials(h2, sh)
    xe2 = _combine_scale(pa2, degB)
    pb2 = _seg_sum_partials(xe2, hs)
    return _combine_final(pb2, degD, b2)


# speedup vs baseline: 12.5820x; 1.3777x over previous
import functools

import jax
import jax.numpy as jnp
from jax import lax
from jax.experimental import pallas as pl
from jax.experimental.pallas import tpu as pltpu
from jax.experimental.pallas import tpu_sc as plsc

N_NODES = 10000
N_HE = 10000
N_EDGES = 320000
DIM = 128

NC = 2
NS = 16
NW = NC * NS
E_PER_TILE = N_EDGES // NW
CHUNK = 80
N_CHUNKS = E_PER_TILE // CHUNK
K = 2
NB = N_CHUNKS // K
KC = 5
NBC = N_CHUNKS // KC
N_PAD = 10240
ROWS_PER_TILE = N_PAD // NS
ZROWS = 32

_MESH = dict(core_axis_name="c", subcore_axis_name="s")


def _seg_sum_partials(values, idx3):
    mesh = plsc.VectorSubcoreMesh(**_MESH)

    @functools.partial(
        pl.kernel,
        mesh=mesh,
        out_type=jax.ShapeDtypeStruct((NC, N_PAD, DIM), jnp.float32),
        scratch_types=(
            [pltpu.VMEM((2, CHUNK), jnp.int32) for _ in range(2)]
            + [pltpu.VMEM((CHUNK, DIM), jnp.float32) for _ in range(2)]
            + [pltpu.VMEM_SHARED((N_PAD, DIM), jnp.float32)]
            + [pltpu.SemaphoreType.DMA for _ in range(6)]
        ),
    )
    def k(vals_hbm, idx_hbm, out_hbm, *refs):
        idxb = refs[0:2]
        rows = refs[2:4]
        acc = refs[4]
        isem = refs[5:7]
        gsem = refs[7:9]
        ssem = refs[9:11]
        cid = lax.axis_index("c")
        sid = lax.axis_index("s")
        wid = sid * NC + cid

        def zrow(r, _):
            for c in range(DIM // 16):
                rows[0][r, pl.ds(c * 16, 16)] = jnp.zeros((16,), jnp.float32)
            return 0

        lax.fori_loop(0, CHUNK, zrow, 0)
        row0 = sid * ROWS_PER_TILE

        def zcp(j, _):
            pltpu.sync_copy(rows[0], acc.at[pl.ds(row0 + j * CHUNK, CHUNK)])
            return 0

        lax.fori_loop(0, ROWS_PER_TILE // CHUNK, zcp, 0)
        plsc.subcore_barrier()

        def issue_i(p, c):
            pltpu.async_copy(idx_hbm.at[wid, c], idxb[p], isem[p])

        def drain_i(p):
            pltpu.make_async_copy(idx_hbm.at[wid, 0], idxb[p], isem[p]).wait()

        def issue_g(p, _c):
            pltpu.async_copy(vals_hbm.at[idxb[p].at[0]], rows[p], gsem[p])

        def drain_g(p):
            pltpu.make_async_copy(vals_hbm.at[idxb[0].at[0]], rows[p],
                                  gsem[p]).wait()

        def issue_s(p, _c):
            pltpu.async_copy(rows[p], acc.at[idxb[p].at[1]], ssem[p], add=True)

        def drain_s(p):
            pltpu.make_async_copy(rows[p], acc.at[idxb[0].at[1]],
                                  ssem[p]).wait()

        A, B = 0, 1

        issue_i(A, 0)
        drain_i(A)
        issue_g(A, 0)
        issue_i(B, 1)
        drain_i(B)
        issue_g(B, 1)
        drain_g(A)
        issue_s(A, 0)
        drain_s(A)
        issue_i(A, 2)
        drain_g(B)
        issue_s(B, 1)
        drain_i(A)
        issue_g(A, 2)

        def body(j, _):
            cA = 2 * j
            drain_s(B)
            issue_i(B, cA + 1)
            drain_g(A)
            issue_s(A, cA)
            drain_i(B)
            issue_g(B, cA + 1)
            drain_s(A)
            issue_i(A, cA + 2)
            drain_g(B)
            issue_s(B, cA + 1)
            drain_i(A)
            issue_g(A, cA + 2)
            return 0

        lax.fori_loop(1, (N_CHUNKS - 1) // 2, body, 0)

        drain_s(B)
        drain_g(A)
        issue_s(A, N_CHUNKS - 1)
        drain_s(A)

        plsc.subcore_barrier()

        def wb(j, _):
            r0 = row0 + j * CHUNK
            pltpu.sync_copy(acc.at[pl.ds(r0, CHUNK)], rows[0])
            pltpu.sync_copy(rows[0], out_hbm.at[cid, pl.ds(r0, CHUNK)])
            return 0

        lax.fori_loop(0, ROWS_PER_TILE // CHUNK, wb, 0)

    return k(values, idx3)


def _count_partials(sidx3):
    mesh = plsc.VectorSubcoreMesh(**_MESH)

    @functools.partial(
        pl.kernel,
        mesh=mesh,
        out_type=jax.ShapeDtypeStruct((NC, N_PAD, DIM), jnp.float32),
        scratch_types=(
            [pltpu.VMEM((N_CHUNKS, CHUNK), jnp.int32),
             pltpu.VMEM((CHUNK, DIM), jnp.float32),
             pltpu.VMEM((ZROWS, DIM), jnp.float32),
             pltpu.VMEM_SHARED((N_PAD, DIM), jnp.float32)]
            + [pltpu.SemaphoreType.DMA for _ in range(KC)]
        ),
    )
    def k(sidx_hbm, out_hbm, sidx2d, ones, zbuf, acc, *sems):
        cid = lax.axis_index("c")
        sid = lax.axis_index("s")
        wid = sid * NC + cid

        def fill(r, _):
            for c in range(DIM // 16):
                ones[r, pl.ds(c * 16, 16)] = jnp.ones((16,), jnp.float32)
            return 0

        lax.fori_loop(0, CHUNK, fill, 0)

        def zrow(r, _):
            for c in range(DIM // 16):
                zbuf[r, pl.ds(c * 16, 16)] = jnp.zeros((16,), jnp.float32)
            return 0

        lax.fori_loop(0, ZROWS, zrow, 0)
        row0 = sid * ROWS_PER_TILE

        def zcp(j, _):
            pltpu.sync_copy(zbuf, acc.at[pl.ds(row0 + j * ZROWS, ZROWS)])
            return 0

        lax.fori_loop(0, ROWS_PER_TILE // ZROWS, zcp, 0)

        pltpu.sync_copy(sidx_hbm.at[wid], sidx2d)
        plsc.subcore_barrier()

        def issue(c, p):
            pltpu.async_copy(ones, acc.at[sidx2d.at[c]], sems[p], add=True)

        def drain(p):
            pltpu.make_async_copy(ones, acc.at[sidx2d.at[0]], sems[p]).wait()

        for p in range(KC):
            issue(p, p)

        def body(j, _):
            for p in range(KC):
                drain(p)
                issue(j * KC + p, p)
            return 0

        lax.fori_loop(1, NBC, body, 0)
        for p in range(KC):
            drain(p)

        plsc.subcore_barrier()

        def wb(j, _):
            r0 = row0 + j * ZROWS
            pltpu.sync_copy(acc.at[pl.ds(r0, ZROWS)], zbuf)
            pltpu.sync_copy(zbuf, out_hbm.at[cid, pl.ds(r0, ZROWS)])
            return 0

        lax.fori_loop(0, ROWS_PER_TILE // ZROWS, wb, 0)

    return k(sidx3)


BLK = 400


def _matmul(h, W):
    def body(h_ref, w_ref, o_ref):
        o_ref[...] = jnp.dot(h_ref[...], w_ref[...],
                             preferred_element_type=jnp.float32)

    return pl.pallas_call(
        body,
        grid=(N_NODES // BLK,),
        in_specs=[
            pl.BlockSpec((BLK, DIM), lambda i: (i, 0)),
            pl.BlockSpec((DIM, DIM), lambda i: (0, 0)),
        ],
        out_specs=pl.BlockSpec((BLK, DIM), lambda i: (i, 0)),
        out_shape=jax.ShapeDtypeStruct((N_NODES, DIM), jnp.float32),
    )(h, W)


def _inv_deg(d_ref):
    d = d_ref[0, :, 0] + d_ref[1, :, 0]
    return jnp.where(d > 0.0, 1.0 / d, 0.0)


def _combine_scale(parts, deg):
    def body(p_ref, d_ref, o_ref):
        inv = _inv_deg(d_ref)
        o_ref[...] = inv[:, None] * (p_ref[0] + p_ref[1])

    return pl.pallas_call(
        body,
        grid=(N_NODES // BLK,),
        in_specs=[
            pl.BlockSpec((NC, BLK, DIM), lambda i: (0, i, 0)),
            pl.BlockSpec((NC, BLK, DIM), lambda i: (0, i, 0)),
        ],
        out_specs=pl.BlockSpec((BLK, DIM), lambda i: (i, 0)),
        out_shape=jax.ShapeDtypeStruct((N_NODES, DIM), jnp.float32),
    )(parts, deg)


def _combine_mm(parts, deg, b, W):
    def body(p_ref, d_ref, b_ref, w_ref, o_ref):
        inv = _inv_deg(d_ref)
        t = inv[:, None] * (p_ref[0] + p_ref[1]) + b_ref[...][None, :]
        t = jnp.where(t > 0.0, t, jnp.exp(jnp.minimum(t, 0.0)) - 1.0)
        o_ref[...] = jnp.dot(t, w_ref[...], preferred_element_type=jnp.float32)

    return pl.pallas_call(
        body,
        grid=(N_NODES // BLK,),
        in_specs=[
            pl.BlockSpec((NC, BLK, DIM), lambda i: (0, i, 0)),
            pl.BlockSpec((NC, BLK, DIM), lambda i: (0, i, 0)),
            pl.BlockSpec((DIM,), lambda i: (0,)),
            pl.BlockSpec((DIM, DIM), lambda i: (0, 0)),
        ],
        out_specs=pl.BlockSpec((BLK, DIM), lambda i: (i, 0)),
        out_shape=jax.ShapeDtypeStruct((N_NODES, DIM), jnp.float32),
    )(parts, deg, b, W)


def _combine_final(parts, deg, b):
    def body(p_ref, d_ref, b_ref, o_ref):
        inv = _inv_deg(d_ref)
        o_ref[...] = inv[:, None] * (p_ref[0] + p_ref[1]) + b_ref[...][None, :]

    return pl.pallas_call(
        body,
        grid=(N_NODES // BLK,),
        in_specs=[
            pl.BlockSpec((NC, BLK, DIM), lambda i: (0, i, 0)),
            pl.BlockSpec((NC, BLK, DIM), lambda i: (0, i, 0)),
            pl.BlockSpec((DIM,), lambda i: (0,)),
        ],
        out_specs=pl.BlockSpec((BLK, DIM), lambda i: (i, 0)),
        out_shape=jax.ShapeDtypeStruct((N_NODES, DIM), jnp.float32),
    )(parts, deg, b)


def kernel(x, edges, edge_weight, W1, b1, W2, b2):
    del edge_weight
    src3 = edges[0].reshape(NW, N_CHUNKS, CHUNK)
    he3 = edges[1].reshape(NW, N_CHUNKS, CHUNK)
    sh = jnp.stack([src3, he3], axis=2)
    hs = jnp.stack([he3, src3], axis=2)

    degD = _count_partials(src3)
    degB = _count_partials(he3)

    h = _matmul(x, W1)
    pa = _seg_sum_partials(h, sh)
    xe = _combine_scale(pa, degB)
    pb = _seg_sum_partials(xe, hs)
    h2 = _combine_mm(pb, degD, b1, W2)

    pa2 = _seg_sum_partials(h2, sh)
    xe2 = _combine_scale(pa2, degB)
    pb2 = _seg_sum_partials(xe2, hs)
    return _combine_final(pb2, degD, b2)


# speedup vs baseline: 12.7458x; 1.0130x over previous
import functools

import jax
import jax.numpy as jnp
from jax import lax
from jax.experimental import pallas as pl
from jax.experimental.pallas import tpu as pltpu
from jax.experimental.pallas import tpu_sc as plsc

N_NODES = 10000
N_HE = 10000
N_EDGES = 320000
DIM = 128

NC = 2
NS = 16
NW = NC * NS
E_PER_TILE = N_EDGES // NW
CHUNK = 80
N_CHUNKS = E_PER_TILE // CHUNK
K = 2
NB = N_CHUNKS // K
KC = 5
NBC = N_CHUNKS // KC
N_PAD = 10240
ROWS_PER_TILE = N_PAD // NS
ZROWS = 32

_MESH = dict(core_axis_name="c", subcore_axis_name="s")


def _seg_sum_partials(values, idx3):
    mesh = plsc.VectorSubcoreMesh(**_MESH)

    @functools.partial(
        pl.kernel,
        mesh=mesh,
        out_type=jax.ShapeDtypeStruct((NC, N_PAD, DIM), jnp.float32),
        scratch_types=(
            [pltpu.VMEM((2, CHUNK), jnp.int32) for _ in range(2)]
            + [pltpu.VMEM((CHUNK, DIM), jnp.float32) for _ in range(2)]
            + [pltpu.VMEM_SHARED((N_PAD, DIM), jnp.float32)]
            + [pltpu.SemaphoreType.DMA for _ in range(6)]
        ),
    )
    def k(vals_hbm, idx_hbm, out_hbm, *refs):
        idxb = refs[0:2]
        rows = refs[2:4]
        acc = refs[4]
        isem = refs[5:7]
        gsem = refs[7:9]
        ssem = refs[9:11]
        cid = lax.axis_index("c")
        sid = lax.axis_index("s")
        wid = sid * NC + cid

        row0 = sid * ROWS_PER_TILE

        def issue_i(p, c):
            pltpu.async_copy(idx_hbm.at[wid, c], idxb[p], isem[p])

        def drain_i(p):
            pltpu.make_async_copy(idx_hbm.at[wid, 0], idxb[p], isem[p]).wait()

        def issue_g(p, _c):
            pltpu.async_copy(vals_hbm.at[idxb[p].at[0]], rows[p], gsem[p])

        def drain_g(p):
            pltpu.make_async_copy(vals_hbm.at[idxb[0].at[0]], rows[p],
                                  gsem[p]).wait()

        def issue_s(p, _c):
            pltpu.async_copy(rows[p], acc.at[idxb[p].at[1]], ssem[p], add=True)

        def drain_s(p):
            pltpu.make_async_copy(rows[p], acc.at[idxb[0].at[1]],
                                  ssem[p]).wait()

        A, B = 0, 1

        issue_i(A, 0)
        issue_i(B, 1)
        drain_i(A)
        issue_g(A, 0)

        def zrow(r, _):
            for c in range(DIM // 16):
                rows[1][r, pl.ds(c * 16, 16)] = jnp.zeros((16,), jnp.float32)
            return 0

        lax.fori_loop(0, CHUNK, zrow, 0)

        def zcp(j, _):
            pltpu.sync_copy(rows[1], acc.at[pl.ds(row0 + j * CHUNK, CHUNK)])
            return 0

        lax.fori_loop(0, ROWS_PER_TILE // CHUNK, zcp, 0)
        plsc.subcore_barrier()

        drain_i(B)
        issue_g(B, 1)
        drain_g(A)
        issue_s(A, 0)
        drain_s(A)
        issue_i(A, 2)
        drain_g(B)
        issue_s(B, 1)
        drain_i(A)
        issue_g(A, 2)

        def body(j, _):
            cA = 2 * j
            drain_s(B)
            issue_i(B, cA + 1)
            drain_g(A)
            issue_s(A, cA)
            drain_i(B)
            issue_g(B, cA + 1)
            drain_s(A)
            issue_i(A, cA + 2)
            drain_g(B)
            issue_s(B, cA + 1)
            drain_i(A)
            issue_g(A, cA + 2)
            return 0

        lax.fori_loop(1, (N_CHUNKS - 1) // 2, body, 0)

        drain_s(B)
        drain_g(A)
        issue_s(A, N_CHUNKS - 1)
        drain_s(A)

        plsc.subcore_barrier()

        NWB = ROWS_PER_TILE // CHUNK

        def rd(c, p):
            pltpu.async_copy(acc.at[pl.ds(row0 + c * CHUNK, CHUNK)],
                             rows[p], gsem[p])

        def wr(c, p):
            pltpu.async_copy(rows[p],
                             out_hbm.at[cid, pl.ds(row0 + c * CHUNK, CHUNK)],
                             ssem[p])

        rd(0, 0)
        for c in range(NWB):
            p = c % 2
            pltpu.make_async_copy(acc.at[pl.ds(row0, CHUNK)], rows[p],
                                  gsem[p]).wait()
            wr(c, p)
            if c + 1 < NWB:
                if c >= 1:
                    pltpu.make_async_copy(
                        rows[1 - p], out_hbm.at[cid, pl.ds(row0, CHUNK)],
                        ssem[1 - p]).wait()
                rd(c + 1, 1 - p)
        pltpu.make_async_copy(rows[0], out_hbm.at[cid, pl.ds(row0, CHUNK)],
                              ssem[0]).wait()
        pltpu.make_async_copy(rows[1], out_hbm.at[cid, pl.ds(row0, CHUNK)],
                              ssem[1]).wait()

    return k(values, idx3)


def _count_partials(sidx3):
    mesh = plsc.VectorSubcoreMesh(**_MESH)

    @functools.partial(
        pl.kernel,
        mesh=mesh,
        out_type=jax.ShapeDtypeStruct((NC, N_PAD, DIM), jnp.float32),
        scratch_types=(
            [pltpu.VMEM((N_CHUNKS, CHUNK), jnp.int32),
             pltpu.VMEM((CHUNK, DIM), jnp.float32),
             pltpu.VMEM((ZROWS, DIM), jnp.float32),
             pltpu.VMEM_SHARED((N_PAD, DIM), jnp.float32)]
            + [pltpu.SemaphoreType.DMA for _ in range(KC)]
        ),
    )
    def k(sidx_hbm, out_hbm, sidx2d, ones, zbuf, acc, *sems):
        cid = lax.axis_index("c")
        sid = lax.axis_index("s")
        wid = sid * NC + cid

        def fill(r, _):
            for c in range(DIM // 16):
                ones[r, pl.ds(c * 16, 16)] = jnp.ones((16,), jnp.float32)
            return 0

        lax.fori_loop(0, CHUNK, fill, 0)

        def zrow(r, _):
            for c in range(DIM // 16):
                zbuf[r, pl.ds(c * 16, 16)] = jnp.zeros((16,), jnp.float32)
            return 0

        lax.fori_loop(0, ZROWS, zrow, 0)
        row0 = sid * ROWS_PER_TILE

        def zcp(j, _):
            pltpu.sync_copy(zbuf, acc.at[pl.ds(row0 + j * ZROWS, ZROWS)])
            return 0

        lax.fori_loop(0, ROWS_PER_TILE // ZROWS, zcp, 0)

        pltpu.sync_copy(sidx_hbm.at[wid], sidx2d)
        plsc.subcore_barrier()

        def issue(c, p):
            pltpu.async_copy(ones, acc.at[sidx2d.at[c]], sems[p], add=True)

        def drain(p):
            pltpu.make_async_copy(ones, acc.at[sidx2d.at[0]], sems[p]).wait()

        for p in range(KC):
            issue(p, p)

        def body(j, _):
            for p in range(KC):
                drain(p)
                issue(j * KC + p, p)
            return 0

        lax.fori_loop(1, NBC, body, 0)
        for p in range(KC):
            drain(p)

        plsc.subcore_barrier()

        def wb(j, _):
            r0 = row0 + j * ZROWS
            pltpu.sync_copy(acc.at[pl.ds(r0, ZROWS)], zbuf)
            pltpu.sync_copy(zbuf, out_hbm.at[cid, pl.ds(r0, ZROWS)])
            return 0

        lax.fori_loop(0, ROWS_PER_TILE // ZROWS, wb, 0)

    return k(sidx3)


BLK = 400


def _matmul(h, W):
    def body(h_ref, w_ref, o_ref):
        o_ref[...] = jnp.dot(h_ref[...], w_ref[...],
                             preferred_element_type=jnp.float32)

    return pl.pallas_call(
        body,
        grid=(N_NODES // BLK,),
        in_specs=[
            pl.BlockSpec((BLK, DIM), lambda i: (i, 0)),
            pl.BlockSpec((DIM, DIM), lambda i: (0, 0)),
        ],
        out_specs=pl.BlockSpec((BLK, DIM), lambda i: (i, 0)),
        out_shape=jax.ShapeDtypeStruct((N_NODES, DIM), jnp.float32),
    )(h, W)


def _inv_deg(d_ref):
    d = d_ref[0, :, 0] + d_ref[1, :, 0]
    return jnp.where(d > 0.0, 1.0 / d, 0.0)


def _combine_scale(parts, deg):
    def body(p_ref, d_ref, o_ref):
        inv = _inv_deg(d_ref)
        o_ref[...] = inv[:, None] * (p_ref[0] + p_ref[1])

    return pl.pallas_call(
        body,
        grid=(N_NODES // BLK,),
        in_specs=[
            pl.BlockSpec((NC, BLK, DIM), lambda i: (0, i, 0)),
            pl.BlockSpec((NC, BLK, DIM), lambda i: (0, i, 0)),
        ],
        out_specs=pl.BlockSpec((BLK, DIM), lambda i: (i, 0)),
        out_shape=jax.ShapeDtypeStruct((N_NODES, DIM), jnp.float32),
    )(parts, deg)


def _combine_mm(parts, deg, b, W):
    def body(p_ref, d_ref, b_ref, w_ref, o_ref):
        inv = _inv_deg(d_ref)
        t = inv[:, None] * (p_ref[0] + p_ref[1]) + b_ref[...][None, :]
        t = jnp.where(t > 0.0, t, jnp.exp(jnp.minimum(t, 0.0)) - 1.0)
        o_ref[...] = jnp.dot(t, w_ref[...], preferred_element_type=jnp.float32)

    return pl.pallas_call(
        body,
        grid=(N_NODES // BLK,),
        in_specs=[
            pl.BlockSpec((NC, BLK, DIM), lambda i: (0, i, 0)),
            pl.BlockSpec((NC, BLK, DIM), lambda i: (0, i, 0)),
            pl.BlockSpec((DIM,), lambda i: (0,)),
            pl.BlockSpec((DIM, DIM), lambda i: (0, 0)),
        ],
        out_specs=pl.BlockSpec((BLK, DIM), lambda i: (i, 0)),
        out_shape=jax.ShapeDtypeStruct((N_NODES, DIM), jnp.float32),
    )(parts, deg, b, W)


def _combine_final(parts, deg, b):
    def body(p_ref, d_ref, b_ref, o_ref):
        inv = _inv_deg(d_ref)
        o_ref[...] = inv[:, None] * (p_ref[0] + p_ref[1]) + b_ref[...][None, :]

    return pl.pallas_call(
        body,
        grid=(N_NODES // BLK,),
        in_specs=[
            pl.BlockSpec((NC, BLK, DIM), lambda i: (0, i, 0)),
            pl.BlockSpec((NC, BLK, DIM), lambda i: (0, i, 0)),
            pl.BlockSpec((DIM,), lambda i: (0,)),
        ],
        out_specs=pl.BlockSpec((BLK, DIM), lambda i: (i, 0)),
        out_shape=jax.ShapeDtypeStruct((N_NODES, DIM), jnp.float32),
    )(parts, deg, b)


def kernel(x, edges, edge_weight, W1, b1, W2, b2):
    del edge_weight
    src3 = edges[0].reshape(NW, N_CHUNKS, CHUNK)
    he3 = edges[1].reshape(NW, N_CHUNKS, CHUNK)
    sh = jnp.stack([src3, he3], axis=2)
    hs = jnp.stack([he3, src3], axis=2)

    degD = _count_partials(src3)
    degB = _count_partials(he3)

    h = _matmul(x, W1)
    pa = _seg_sum_partials(h, sh)
    xe = _combine_scale(pa, degB)
    pb = _seg_sum_partials(xe, hs)
    h2 = _combine_mm(pb, degD, b1, W2)

    pa2 = _seg_sum_partials(h2, sh)
    xe2 = _combine_scale(pa2, degB)
    pb2 = _seg_sum_partials(xe2, hs)
    return _combine_final(pb2, degD, b2)
